# Initial kernel scaffold; baseline (speedup 1.0000x reference)
import jax
import jax.numpy as jnp
from jax.experimental import pallas as pl


def kernel(x, table, W, b):
    raise NotImplementedError("write your pallas kernel here")


# speedup vs baseline: 13.0611x; 13.0611x over previous
import functools

import jax
import jax.numpy as jnp
from jax import lax
from jax.experimental import pallas as pl
from jax.experimental.pallas import tpu as pltpu
from jax.experimental.pallas import tpu_sc as plsc

VOCAB = 1000000
EMBED = 32
BATCH = 16384
SEQ = 200

ROWS4 = VOCAB // 4
TC_BLK = 25000
LANES = 16
GROUP = 16
NGROUPS = BATCH // GROUP
IDX_PER_GROUP = GROUP * SEQ
IDX_ROWS = IDX_PER_GROUP // 128


def _tc_matvec_body(t_ref, wg_ref, o_ref):
    o_ref[...] = jnp.dot(t_ref[...], wg_ref[...],
                         preferred_element_type=jnp.float32)


def _tc_matvec(table4, wg):
    return pl.pallas_call(
        _tc_matvec_body,
        grid=(ROWS4 // TC_BLK,),
        in_specs=[
            pl.BlockSpec((TC_BLK, 128), lambda i: (i, 0)),
            pl.BlockSpec((128, 4), lambda i: (0, 0)),
        ],
        out_specs=pl.BlockSpec((TC_BLK, 4), lambda i: (i, 0)),
        out_shape=jax.ShapeDtypeStruct((ROWS4, 4), jnp.float32),
    )(table4, wg)


def _sc_pool(xt, tw, b16):
    info = plsc.get_sparse_core_info()
    nc, ns = info.num_cores, info.num_subcores
    nw = nc * ns
    per_w = NGROUPS // nw

    @functools.partial(
        pl.kernel,
        out_type=jax.ShapeDtypeStruct((BATCH,), jnp.float32),
        mesh=plsc.VectorSubcoreMesh(core_axis_name="c", subcore_axis_name="s"),
        scratch_types=[
            pltpu.VMEM((IDX_ROWS, 128), jnp.int32),
            pltpu.VMEM((IDX_ROWS, 128), jnp.float32),
            pltpu.VMEM((LANES,), jnp.float32),
            pltpu.VMEM((LANES,), jnp.float32),
            pltpu.SemaphoreType.DMA,
        ],
    )
    def k(xt_hbm, tw_hbm, b_hbm, out_hbm, idx_v, vals_v, b_v, out_v, sem):
        wid = lax.axis_index("s") * nc + lax.axis_index("c")
        pltpu.sync_copy(b_hbm, b_v)

        def per_group(g, carry):
            gg = wid * per_w + g
            pltpu.sync_copy(xt_hbm.at[gg], idx_v)
            cps = []
            for c in range(IDX_ROWS):
                cps.append(pltpu.async_copy(
                    tw_hbm.at[idx_v.at[c]], vals_v.at[c], sem))
            for cp in cps:
                cp.wait()
            acc = jnp.zeros((LANES,), jnp.float32)
            for s in range(SEQ):
                acc = acc + vals_v[s >> 3, pl.ds((s & 7) * LANES, LANES)]
            z = acc * (1.0 / SEQ) + b_v[...]
            out_v[...] = 1.0 / (1.0 + jnp.exp(-z))
            pltpu.sync_copy(out_v, out_hbm.at[pl.ds(gg * GROUP, GROUP)])
            return carry

        lax.fori_loop(0, per_w, per_group, 0)

    return k(xt, tw, b16)


def kernel(x, table, W, b):
    table4 = table.reshape(ROWS4, 128)
    wg = jnp.kron(jnp.eye(4, dtype=jnp.float32), W)
    tw = _tc_matvec(table4, wg).reshape(VOCAB)
    xt = (x.astype(jnp.int32)
           .reshape(NGROUPS, GROUP, SEQ)
           .transpose(0, 2, 1)
           .reshape(NGROUPS, IDX_ROWS, 128))
    b16 = jnp.broadcast_to(b.astype(jnp.float32), (LANES,))
    return _sc_pool(xt, tw, b16)
